# baseline (speedup 1.0000x reference)
import dataclasses
import functools

import jax
import jax.numpy as jnp
from jax import lax
from jax.experimental import pallas as pl
from jax.experimental.pallas import tpu as pltpu
from jax.experimental.pallas import tpu_sc as plsc

_NC = 2
_NS = 16
_NW = _NC * _NS


_COLS = 32768
_HLINES = _COLS // 2
_CSHIFT = _COLS.bit_length() - 1
_QSHIFT = _CSHIFT - 2
_QMASK = _COLS // 4 - 1
_D_HALF = 32


def _pack_pairs(gama_t):
    D, N = gama_t.shape
    grid = (N + _COLS - 1) // _COLS

    def body(x_ref, o_ref):
        xt = x_ref[...].T
        xb = xt.astype(jnp.bfloat16)
        lo = lax.bitcast_convert_type(xb[:, : _D_HALF], jnp.uint16)
        hi = lax.bitcast_convert_type(xb[:, _D_HALF:], jnp.uint16)
        w = (hi.astype(jnp.uint32) << 16) | lo.astype(jnp.uint32)
        wf = lax.bitcast_convert_type(w, jnp.float32)
        q = _COLS // 4
        o_ref[...] = jnp.concatenate(
            [wf[:q], wf[q:2 * q], wf[2 * q:3 * q], wf[3 * q:]], axis=1)

    return pl.pallas_call(
        body,
        grid=(grid,),
        in_specs=[pl.BlockSpec((D, _COLS), lambda j: (0, j))],
        out_specs=pl.BlockSpec((_COLS // 4, 2 * D), lambda j: (j, 0)),
        out_shape=jax.ShapeDtypeStruct((grid * (_COLS // 4), 2 * D),
                                       jnp.float32),
        compiler_params=pltpu.CompilerParams(
            dimension_semantics=("arbitrary",)),
    )(gama_t)


def _sc_compiler_params():
    cp = pltpu.CompilerParams()
    if "needs_layout_passes" in pltpu.CompilerParams.__dataclass_fields__:
        cp = dataclasses.replace(cp, needs_layout_passes=False)
    return cp


def _sc_gather_dot(users, items, ugp, igp, ub_flat, ib_flat):
    B = users.shape[0]
    D2 = ugp.shape[1]
    D = D2 // 2
    bw = B // _NW
    HALF = bw // 2
    mesh = plsc.VectorSubcoreMesh(core_axis_name="c", subcore_axis_name="s")

    @functools.partial(
        pl.kernel,
        mesh=mesh,
        compiler_params=_sc_compiler_params(),
        out_type=jax.ShapeDtypeStruct((B,), jnp.float32),
        scratch_types=[
            pltpu.VMEM((bw,), jnp.int32),
            pltpu.VMEM((bw,), jnp.int32),
            pltpu.VMEM((bw,), jnp.int32),
            pltpu.VMEM((bw,), jnp.int32),
            pltpu.VMEM((HALF, 128), jnp.float32),
            pltpu.VMEM((HALF, 128), jnp.float32),
            pltpu.VMEM((bw,), jnp.float32),
            pltpu.VMEM((bw,), jnp.float32),
            pltpu.VMEM((bw,), jnp.float32),
            pltpu.SemaphoreType.DMA,
            pltpu.SemaphoreType.DMA,
        ],
    )
    def k(u_hbm, i_hbm, ugp_hbm, igp_hbm, ub_hbm, ib_hbm, out_hbm,
          u_v, i_v, ul_v, il_v, ug_v, ig_v, ub_v, ib_v, o_v, sem, bsem):
        wid = lax.axis_index("s") * _NC + lax.axis_index("c")
        base = wid * bw
        pltpu.sync_copy(u_hbm.at[pl.ds(base, bw)], u_v)
        pltpu.sync_copy(i_hbm.at[pl.ds(base, bw)], i_v)

        cb1 = pltpu.async_copy(ub_hbm.at[u_v], ub_v, bsem)
        cb2 = pltpu.async_copy(ib_hbm.at[i_v], ib_v, bsem)

        @pl.loop(0, bw, step=16)
        def _(t):
            s = pl.ds(t, 16)
            u = u_v[s]
            i = i_v[s]
            ul_v[s] = ((u >> _CSHIFT) << _QSHIFT) | (u & _QMASK)
            il_v[s] = ((i >> _CSHIFT) << _QSHIFT) | (i & _QMASK)

        j16 = lax.iota(jnp.int32, 16)

        def do_round(r):
            rbase = r * HALF
            g1 = pltpu.async_copy(ugp_hbm.at[ul_v.at[pl.ds(rbase, HALF)]],
                                  ug_v, sem)
            g2 = pltpu.async_copy(igp_hbm.at[il_v.at[pl.ds(rbase, HALF)]],
                                  ig_v, sem)
            g1.wait()
            g2.wait()

            @pl.loop(0, HALF, step=16)
            def _(t):
                s = pl.ds(rbase + t, 16)
                brow = j16 + t
                upar = ((u_v[s] >> _QSHIFT) & 3) * _D_HALF
                ipar = ((i_v[s] >> _QSHIFT) & 3) * _D_HALF
                acc = jnp.zeros((16,), jnp.float32)

                def jbody(j, a):
                    wu = plsc.load_gather(ug_v, [brow, upar + j])
                    wi = plsc.load_gather(ig_v, [brow, ipar + j])
                    ua, ub = plsc.unpack(plsc.bitcast(wu, jnp.bfloat16),
                                         format=plsc.PackFormat.INTERLEAVED)
                    ia, ib = plsc.unpack(plsc.bitcast(wi, jnp.bfloat16),
                                         format=plsc.PackFormat.INTERLEAVED)
                    return a + ua * ia + ub * ib

                acc = lax.fori_loop(0, _D_HALF, jbody, acc)
                o_v[s] = acc

        do_round(0)
        do_round(1)

        cb1.wait()
        cb2.wait()

        @pl.loop(0, bw, step=16)
        def _(t):
            s = pl.ds(t, 16)
            o_v[s] = o_v[s] + ub_v[s] + ib_v[s]

        pltpu.sync_copy(o_v, out_hbm.at[pl.ds(base, bw)])

    return k(users, items, ugp, igp, ub_flat, ib_flat)


def _pad128(gama):
    n, d = gama.shape
    return jnp.pad(gama, ((0, 0), (0, 2 * d - gama.shape[1])))


def kernel(users, items, user_gama, item_gama, user_beta, item_beta):
    users = users.astype(jnp.int32)
    items = items.astype(jnp.int32)
    ugp = _pack_pairs(user_gama.T)
    igp = _pack_pairs(item_gama.T)
    return _sc_gather_dot(users, items, ugp, igp,
                          user_beta.reshape(-1), item_beta.reshape(-1))


# speedup vs baseline: 1.3216x; 1.3216x over previous
import dataclasses
import functools

import jax
import jax.numpy as jnp
from jax import lax
from jax.experimental import pallas as pl
from jax.experimental.pallas import tpu as pltpu
from jax.experimental.pallas import tpu_sc as plsc

_NC = 2
_NS = 16
_NW = _NC * _NS


_COLS = 32768
_HLINES = _COLS // 2
_CSHIFT = _COLS.bit_length() - 1
_HMASK = _HLINES - 1


def _pack_pairs(gama_t):
    D, N = gama_t.shape
    grid = (N + _COLS - 1) // _COLS

    def body(x_ref, o_ref):
        xt = x_ref[...].T
        o_ref[:, :D] = xt[:_HLINES]
        o_ref[:, D:] = xt[_HLINES:]

    return pl.pallas_call(
        body,
        grid=(grid,),
        in_specs=[pl.BlockSpec((D, _COLS), lambda j: (0, j))],
        out_specs=pl.BlockSpec((_HLINES, 2 * D), lambda j: (j, 0)),
        out_shape=jax.ShapeDtypeStruct((grid * _HLINES, 2 * D), jnp.float32),
        compiler_params=pltpu.CompilerParams(
            dimension_semantics=("arbitrary",)),
    )(gama_t)


def _sc_compiler_params():
    cp = pltpu.CompilerParams()
    if "needs_layout_passes" in pltpu.CompilerParams.__dataclass_fields__:
        cp = dataclasses.replace(cp, needs_layout_passes=False)
    return cp


def _sc_gather_dot(users, items, ugp, igp, ub_flat, ib_flat):
    B = users.shape[0]
    D2 = ugp.shape[1]
    D = D2 // 2
    bw = B // _NW
    HALF = bw // 2
    mesh = plsc.VectorSubcoreMesh(core_axis_name="c", subcore_axis_name="s")

    @functools.partial(
        pl.kernel,
        mesh=mesh,
        compiler_params=_sc_compiler_params(),
        out_type=jax.ShapeDtypeStruct((B,), jnp.float32),
        scratch_types=[
            pltpu.VMEM((bw,), jnp.int32),
            pltpu.VMEM((bw,), jnp.int32),
            pltpu.VMEM((bw,), jnp.int32),
            pltpu.VMEM((bw,), jnp.int32),
            pltpu.VMEM((HALF, 128), jnp.float32),
            pltpu.VMEM((HALF, 128), jnp.float32),
            pltpu.VMEM((bw,), jnp.float32),
            pltpu.VMEM((bw,), jnp.float32),
            pltpu.VMEM((bw,), jnp.float32),
            pltpu.SemaphoreType.DMA,
            pltpu.SemaphoreType.DMA,
        ],
    )
    def k(u_hbm, i_hbm, ugp_hbm, igp_hbm, ub_hbm, ib_hbm, out_hbm,
          u_v, i_v, ul_v, il_v, ug_v, ig_v, ub_v, ib_v, o_v, sem, bsem):
        wid = lax.axis_index("s") * _NC + lax.axis_index("c")
        base = wid * bw
        pltpu.sync_copy(u_hbm.at[pl.ds(base, bw)], u_v)
        pltpu.sync_copy(i_hbm.at[pl.ds(base, bw)], i_v)

        cb1 = pltpu.async_copy(ub_hbm.at[u_v], ub_v, bsem)
        cb2 = pltpu.async_copy(ib_hbm.at[i_v], ib_v, bsem)

        @pl.loop(0, bw, step=16)
        def _(t):
            s = pl.ds(t, 16)
            u = u_v[s]
            i = i_v[s]
            ul_v[s] = ((u >> _CSHIFT) << (_CSHIFT - 1)) | (u & _HMASK)
            il_v[s] = ((i >> _CSHIFT) << (_CSHIFT - 1)) | (i & _HMASK)

        j16 = lax.iota(jnp.int32, 16)

        def do_round(r):
            rbase = r * HALF
            g1 = pltpu.async_copy(ugp_hbm.at[ul_v.at[pl.ds(rbase, HALF)]],
                                  ug_v, sem)
            g2 = pltpu.async_copy(igp_hbm.at[il_v.at[pl.ds(rbase, HALF)]],
                                  ig_v, sem)
            g1.wait()
            g2.wait()

            @pl.loop(0, HALF, step=16)
            def _(t):
                s = pl.ds(rbase + t, 16)
                brow = j16 + t
                upar = ((u_v[s] >> (_CSHIFT - 1)) & 1) * D
                ipar = ((i_v[s] >> (_CSHIFT - 1)) & 1) * D
                acc = jnp.zeros((16,), jnp.float32)

                def jbody(j, a):
                    uv = plsc.load_gather(ug_v, [brow, upar + j])
                    iv = plsc.load_gather(ig_v, [brow, ipar + j])
                    return a + uv * iv

                acc = lax.fori_loop(0, D, jbody, acc)
                o_v[s] = acc

        do_round(0)
        do_round(1)

        cb1.wait()
        cb2.wait()

        @pl.loop(0, bw, step=16)
        def _(t):
            s = pl.ds(t, 16)
            o_v[s] = o_v[s] + ub_v[s] + ib_v[s]

        pltpu.sync_copy(o_v, out_hbm.at[pl.ds(base, bw)])

    return k(users, items, ugp, igp, ub_flat, ib_flat)


def _pad128(gama):
    n, d = gama.shape
    return jnp.pad(gama, ((0, 0), (0, 2 * d - gama.shape[1])))


def kernel(users, items, user_gama, item_gama, user_beta, item_beta):
    users = users.astype(jnp.int32)
    items = items.astype(jnp.int32)
    ugp = _pack_pairs(user_gama.T)
    igp = _pack_pairs(item_gama.T)
    return _sc_gather_dot(users, items, ugp, igp,
                          user_beta.reshape(-1), item_beta.reshape(-1))


# speedup vs baseline: 1.3252x; 1.0027x over previous
import dataclasses
import functools

import jax
import jax.numpy as jnp
from jax import lax
from jax.experimental import pallas as pl
from jax.experimental.pallas import tpu as pltpu
from jax.experimental.pallas import tpu_sc as plsc

_NC = 2
_NS = 16
_NW = _NC * _NS


_COLS = 32768
_HLINES = _COLS // 2
_CSHIFT = _COLS.bit_length() - 1
_HMASK = _HLINES - 1


def _pack_pairs(gama_t):
    D, N = gama_t.shape
    grid = (N + _COLS - 1) // _COLS

    def body(x_ref, o_ref):
        xt = x_ref[...].T
        o_ref[:, :D] = xt[:_HLINES]
        o_ref[:, D:] = xt[_HLINES:]

    return pl.pallas_call(
        body,
        grid=(grid,),
        in_specs=[pl.BlockSpec((D, _COLS), lambda j: (0, j))],
        out_specs=pl.BlockSpec((_HLINES, 2 * D), lambda j: (j, 0)),
        out_shape=jax.ShapeDtypeStruct((grid * _HLINES, 2 * D), jnp.float32),
        compiler_params=pltpu.CompilerParams(
            dimension_semantics=("arbitrary",)),
    )(gama_t)


def _sc_compiler_params():
    cp = pltpu.CompilerParams()
    if "needs_layout_passes" in pltpu.CompilerParams.__dataclass_fields__:
        cp = dataclasses.replace(cp, needs_layout_passes=False)
    return cp


def _sc_gather_dot(users, items, ugp, igp, ub_flat, ib_flat):
    B = users.shape[0]
    D2 = ugp.shape[1]
    D = D2 // 2
    bw = B // _NW
    HALF = bw // 2
    mesh = plsc.VectorSubcoreMesh(core_axis_name="c", subcore_axis_name="s")

    @functools.partial(
        pl.kernel,
        mesh=mesh,
        compiler_params=_sc_compiler_params(),
        out_type=jax.ShapeDtypeStruct((B,), jnp.float32),
        scratch_types=[
            pltpu.VMEM((bw,), jnp.int32),
            pltpu.VMEM((bw,), jnp.int32),
            pltpu.VMEM((bw,), jnp.int32),
            pltpu.VMEM((bw,), jnp.int32),
            pltpu.VMEM((HALF, 128), jnp.float32),
            pltpu.VMEM((HALF, 128), jnp.float32),
            pltpu.VMEM((bw,), jnp.float32),
            pltpu.VMEM((bw,), jnp.float32),
            pltpu.VMEM((bw,), jnp.float32),
            pltpu.SemaphoreType.DMA,
            pltpu.SemaphoreType.DMA,
        ],
    )
    def k(u_hbm, i_hbm, ugp_hbm, igp_hbm, ub_hbm, ib_hbm, out_hbm,
          u_v, i_v, ul_v, il_v, ug_v, ig_v, ub_v, ib_v, o_v, sem, bsem):
        wid = lax.axis_index("s") * _NC + lax.axis_index("c")
        base = wid * bw
        pltpu.sync_copy(u_hbm.at[pl.ds(base, bw)], u_v)
        pltpu.sync_copy(i_hbm.at[pl.ds(base, bw)], i_v)

        cb1 = pltpu.async_copy(ub_hbm.at[u_v], ub_v, bsem)
        cb2 = pltpu.async_copy(ib_hbm.at[i_v], ib_v, bsem)

        @pl.loop(0, bw, step=16)
        def _(t):
            s = pl.ds(t, 16)
            u = u_v[s]
            i = i_v[s]
            ul_v[s] = ((u >> _CSHIFT) << (_CSHIFT - 1)) | (u & _HMASK)
            il_v[s] = ((i >> _CSHIFT) << (_CSHIFT - 1)) | (i & _HMASK)

        j16 = lax.iota(jnp.int32, 16)

        def do_round(r):
            rbase = r * HALF
            g1 = pltpu.async_copy(ugp_hbm.at[ul_v.at[pl.ds(rbase, HALF)]],
                                  ug_v, sem)
            g2 = pltpu.async_copy(igp_hbm.at[il_v.at[pl.ds(rbase, HALF)]],
                                  ig_v, sem)
            g1.wait()
            g2.wait()

            @pl.loop(0, HALF, step=16)
            def _(t):
                s = pl.ds(rbase + t, 16)
                brow = j16 + t
                upar = ((u_v[s] >> (_CSHIFT - 1)) & 1) * D
                ipar = ((i_v[s] >> (_CSHIFT - 1)) & 1) * D
                acc = jnp.zeros((16,), jnp.float32)

                def jbody(j, a):
                    uv = plsc.load_gather(ug_v, [brow, upar + j])
                    iv = plsc.load_gather(ig_v, [brow, ipar + j])
                    return a + uv * iv

                acc = lax.fori_loop(0, D, jbody, acc)
                o_v[s] = acc

        do_round(0)
        do_round(1)

        cb1.wait()
        cb2.wait()

        @pl.loop(0, bw, step=16)
        def _(t):
            s = pl.ds(t, 16)
            o_v[s] = o_v[s] + ub_v[s] + ib_v[s]

        pltpu.sync_copy(o_v, out_hbm.at[pl.ds(base, bw)])

    return k(users, items, ugp, igp, ub_flat, ib_flat)


def kernel(users, items, user_gama, item_gama, user_beta, item_beta):
    users = users.astype(jnp.int32)
    items = items.astype(jnp.int32)
    ugp = _pack_pairs(user_gama.T)
    igp = _pack_pairs(item_gama.T)
    return _sc_gather_dot(users, items, ugp, igp,
                          user_beta.reshape(-1), item_beta.reshape(-1))
